# baseline (speedup 1.0000x reference)
import functools

import jax
import jax.numpy as jnp
from jax import lax
from jax.experimental import pallas as pl
from jax.experimental.pallas import tpu as pltpu
from jax.experimental.pallas import tpu_sc as plsc

_LANES = 16


@functools.cache
def _build(B, V, D, LP, HALF):
    NW = 32
    S = B // NW
    NBUF = 2

    mesh = plsc.VectorSubcoreMesh(core_axis_name="c", subcore_axis_name="s")

    @functools.partial(
        pl.kernel,
        mesh=mesh,
        compiler_params=pltpu.CompilerParams(
            needs_layout_passes=False, use_tc_tiling_on_sc=False),
        out_type=jax.ShapeDtypeStruct((B,), jnp.float32),
        scratch_types=[
            pltpu.VMEM((S, 2, HALF), jnp.int32),
            pltpu.VMEM((NBUF, LP, D), jnp.float32),
            pltpu.VMEM((S,), jnp.float32),
            pltpu.VMEM((1, D), jnp.float32),
            pltpu.VMEM((2, _LANES), jnp.float32),
            pltpu.SemaphoreType.DMA,
            pltpu.SemaphoreType.DMA,
        ],
    )
    def pooled(src_hbm, table_hbm, wb_hbm, out_hbm,
               idx_v, rows_v, out_v, t0_v, wb_v, sem0, sem1):
        wid = lax.axis_index("s") * 2 + lax.axis_index("c")
        base = wid * S
        sems = (sem0, sem1)

        pltpu.sync_copy(src_hbm.at[pl.ds(base, S)], idx_v)
        pltpu.sync_copy(table_hbm.at[pl.ds(0, 1)], t0_v)
        pltpu.sync_copy(wb_hbm, wb_v)

        def gather_copy(s, buf, h):
            return pltpu.make_async_copy(
                table_hbm.at[idx_v.at[s, h]],
                rows_v.at[buf, pl.ds(h * HALF, HALF)],
                sems[buf],
            )

        def gather_start(s, buf):
            for h in range(2):
                gather_copy(s, buf, h).start()

        def gather_wait(s, buf):
            for h in range(2):
                gather_copy(s, buf, h).wait()

        lanes = lax.iota(jnp.int32, _LANES)
        lane0 = lanes == 0

        def allsum(x):
            for sft in (8, 4, 2, 1):
                x = x + jnp.take_along_axis(x, lanes ^ sft, axis=0)
            return x

        t0v = t0_v[0, :]
        wv = wb_v[0, :]
        bs = allsum(wb_v[1, :])
        lp_f = jnp.float32(LP)
        zero = jnp.zeros((_LANES,), jnp.float32)

        def process(s, buf):
            def sum_body(i, accs):
                a0, a1, a2, a3 = accs
                r = i * 8
                a0 = a0 + rows_v[buf, r, :]
                a1 = a1 + rows_v[buf, r + 1, :]
                a2 = a2 + rows_v[buf, r + 2, :]
                a3 = a3 + rows_v[buf, r + 3, :]
                a0 = a0 + rows_v[buf, r + 4, :]
                a1 = a1 + rows_v[buf, r + 5, :]
                a2 = a2 + rows_v[buf, r + 6, :]
                a3 = a3 + rows_v[buf, r + 7, :]
                return (a0, a1, a2, a3)

            a0, a1, a2, a3 = lax.fori_loop(
                0, LP // 8, sum_body, (zero, zero, zero, zero))
            acc = (a0 + a1) + (a2 + a3)

            macc = zero
            for k in range(LP // _LANES):
                h = k // (HALF // _LANES)
                o = (k % (HALF // _LANES)) * _LANES
                chunk = idx_v[s, h, pl.ds(o, _LANES)]
                macc = macc + jnp.where(chunk != 0, 1.0, 0.0).astype(jnp.float32)
            len_v = allsum(macc)

            corr = acc - (lp_f - len_v) * t0v
            logit_v = allsum(corr * wv) / len_v + bs
            plsc.store_scatter(
                out_v,
                [jnp.broadcast_to(s, (_LANES,)).astype(jnp.int32)],
                logit_v,
                mask=lane0,
            )

        gather_start(0, 0)

        def seq_body(g, carry):
            s0 = 2 * g
            s1 = s0 + 1
            gather_start(s1, 1)
            gather_wait(s0, 0)
            process(s0, 0)
            nxt = lax.rem(s0 + 2, S)
            gather_start(nxt, 0)
            gather_wait(s1, 1)
            process(s1, 1)
            return carry

        lax.fori_loop(0, S // 2, seq_body, 0)
        gather_wait(0, 0)

        for g in range(S // _LANES):
            v = out_v[pl.ds(g * _LANES, _LANES)]
            out_v[pl.ds(g * _LANES, _LANES)] = 1.0 / (1.0 + jnp.exp(-v))

        pltpu.sync_copy(out_v, out_hbm.at[pl.ds(base, S)])

    return pooled


def kernel(src, table, W, b):
    B, L = src.shape
    V, D = table.shape
    LP = 224
    HALF = LP // 2
    src_p = jnp.pad(src, ((0, 0), (0, LP - L))).reshape(B, 2, HALF)
    wb = jnp.concatenate([
        W.reshape(-1).astype(jnp.float32),
        b.reshape(-1).astype(jnp.float32),
        jnp.zeros((_LANES - 1,), jnp.float32),
    ]).reshape(2, _LANES)
    out = _build(B, V, D, LP, HALF)(src_p, table, wb)
    return out.reshape(B, 1)


# speedup vs baseline: 1.8278x; 1.8278x over previous
import functools

import jax
import jax.numpy as jnp
from jax import lax
from jax.experimental import pallas as pl
from jax.experimental.pallas import tpu as pltpu
from jax.experimental.pallas import tpu_sc as plsc

_LANES = 16


@functools.cache
def _build(B, V, D, LP, HALF):
    NW = 32
    S = B // NW
    NBUF = 2

    mesh = plsc.VectorSubcoreMesh(core_axis_name="c", subcore_axis_name="s")

    @functools.partial(
        pl.kernel,
        mesh=mesh,
        compiler_params=pltpu.CompilerParams(
            needs_layout_passes=False, use_tc_tiling_on_sc=False),
        out_type=jax.ShapeDtypeStruct((B,), jnp.float32),
        scratch_types=[
            pltpu.VMEM((S, 2, HALF), jnp.int32),
            pltpu.VMEM((NBUF, LP, D), jnp.float32),
            pltpu.VMEM((S,), jnp.float32),
            pltpu.VMEM((1, D), jnp.float32),
            pltpu.VMEM((2, _LANES), jnp.float32),
            pltpu.SemaphoreType.DMA,
            pltpu.SemaphoreType.DMA,
        ],
    )
    def pooled(src_hbm, table_hbm, wb_hbm, out_hbm,
               idx_v, rows_v, out_v, t0_v, wb_v, sem0, sem1):
        wid = lax.axis_index("s") * 2 + lax.axis_index("c")
        base = wid * S
        sems = (sem0, sem1)

        pltpu.sync_copy(src_hbm.at[pl.ds(base, S)], idx_v)
        pltpu.sync_copy(table_hbm.at[pl.ds(0, 1)], t0_v)
        pltpu.sync_copy(wb_hbm, wb_v)

        def gather_copy(s, buf, h):
            return pltpu.make_async_copy(
                table_hbm.at[idx_v.at[s, h]],
                rows_v.at[buf, pl.ds(h * HALF, HALF)],
                sems[buf],
            )

        def gather_start(s, buf):
            for h in range(2):
                gather_copy(s, buf, h).start()

        def gather_wait(s, buf):
            for h in range(2):
                gather_copy(s, buf, h).wait()

        lanes = lax.iota(jnp.int32, _LANES)
        lane0 = lanes == 0

        def allsum(x):
            for sft in (8, 4, 2, 1):
                x = x + jnp.take_along_axis(x, lanes ^ sft, axis=0)
            return x

        t0v = t0_v[0, :]
        wv = wb_v[0, :]
        bs = allsum(wb_v[1, :])
        lp_f = jnp.float32(LP)
        zero = jnp.zeros((_LANES,), jnp.float32)
        one = jnp.ones((_LANES,), jnp.float32)

        def process(s, buf):
            def sum_body(i, accs):
                a0, a1, a2, a3 = accs
                r = i * 8
                a0 = a0 + rows_v[buf, r, :]
                a1 = a1 + rows_v[buf, r + 1, :]
                a2 = a2 + rows_v[buf, r + 2, :]
                a3 = a3 + rows_v[buf, r + 3, :]
                a0 = a0 + rows_v[buf, r + 4, :]
                a1 = a1 + rows_v[buf, r + 5, :]
                a2 = a2 + rows_v[buf, r + 6, :]
                a3 = a3 + rows_v[buf, r + 7, :]
                return (a0, a1, a2, a3)

            a0, a1, a2, a3 = lax.fori_loop(
                0, LP // 8, sum_body, (zero, zero, zero, zero))
            acc = (a0 + a1) + (a2 + a3)

            macc = zero
            nfull = HALF // _LANES
            rem = HALF - nfull * _LANES
            for h in range(2):
                for k in range(nfull):
                    chunk = idx_v[s, h, pl.ds(k * _LANES, _LANES)]
                    macc = macc + jnp.where(chunk != 0, one, zero)
                if rem:
                    chunk = idx_v[s, h, pl.ds(HALF - _LANES, _LANES)]
                    new = jnp.logical_and(chunk != 0, lanes >= _LANES - rem)
                    macc = macc + jnp.where(new, one, zero)
            len_v = allsum(macc)

            corr = acc - (lp_f - len_v) * t0v
            logit_v = allsum(corr * wv) / len_v + bs
            plsc.store_scatter(
                out_v,
                [jnp.broadcast_to(s, (_LANES,)).astype(jnp.int32)],
                logit_v,
                mask=lane0,
            )

        gather_start(0, 0)

        def seq_body(g, carry):
            s0 = 2 * g
            s1 = s0 + 1
            gather_start(s1, 1)
            gather_wait(s0, 0)
            process(s0, 0)
            nxt = lax.rem(s0 + 2, S)
            gather_start(nxt, 0)
            gather_wait(s1, 1)
            process(s1, 1)
            return carry

        lax.fori_loop(0, S // 2, seq_body, 0)
        gather_wait(0, 0)

        for g in range(S // _LANES):
            v = out_v[pl.ds(g * _LANES, _LANES)]
            out_v[pl.ds(g * _LANES, _LANES)] = 1.0 / (1.0 + jnp.exp(-v))

        pltpu.sync_copy(out_v, out_hbm.at[pl.ds(base, S)])

    return pooled


def kernel(src, table, W, b):
    B, L = src.shape
    V, D = table.shape
    LP = L
    HALF = LP // 2
    src_p = src.reshape(B, 2, HALF)
    wb = jnp.concatenate([
        W.reshape(-1).astype(jnp.float32),
        b.reshape(-1).astype(jnp.float32),
        jnp.zeros((_LANES - 1,), jnp.float32),
    ]).reshape(2, _LANES)
    out = _build(B, V, D, LP, HALF)(src_p, table, wb)
    return out.reshape(B, 1)
